# baseline (speedup 1.0000x reference)
import jax
import jax.numpy as jnp
from jax.experimental import pallas as pl
from jax.experimental.pallas import tpu as pltpu


_BLOCK_ROWS = 2048


def _copy_body(w_hbm, o_ref, sem):
    i = pl.program_id(0)
    copy = pltpu.make_async_copy(
        w_hbm.at[pl.ds(i * _BLOCK_ROWS, _BLOCK_ROWS), :],
        o_ref.at[0],
        sem,
    )
    copy.start()
    copy.wait()


def kernel(x, emb_weight):
    seq_len = x.shape[1]
    dim = emb_weight.shape[1]
    grid = (seq_len // _BLOCK_ROWS,)
    out = pl.pallas_call(
        _copy_body,
        grid=grid,
        out_shape=jax.ShapeDtypeStruct((1, seq_len, dim), emb_weight.dtype),
        in_specs=[pl.BlockSpec(memory_space=pl.ANY)],
        out_specs=pl.BlockSpec((1, _BLOCK_ROWS, dim), lambda i: (0, i, 0)),
        scratch_shapes=[pltpu.SemaphoreType.DMA],
        compiler_params=pltpu.CompilerParams(
            dimension_semantics=("arbitrary",)
        ),
    )(emb_weight)
    return out


# speedup vs baseline: 1.2261x; 1.2261x over previous
import jax
import jax.numpy as jnp
from jax.experimental import pallas as pl
from jax.experimental.pallas import tpu as pltpu


_BLOCK_ROWS = 2048


def _copy_body(w_ref, o_ref):
    o_ref[...] = w_ref[...]


def kernel(x, emb_weight):
    seq_len = x.shape[1]
    dim = emb_weight.shape[1]
    grid = (seq_len // _BLOCK_ROWS,)
    out2d = pl.pallas_call(
        _copy_body,
        grid=grid,
        out_shape=jax.ShapeDtypeStruct((seq_len, dim), emb_weight.dtype),
        in_specs=[pl.BlockSpec((_BLOCK_ROWS, dim), lambda i: (i, 0))],
        out_specs=pl.BlockSpec((_BLOCK_ROWS, dim), lambda i: (i, 0)),
        compiler_params=pltpu.CompilerParams(
            dimension_semantics=("parallel",)
        ),
    )(emb_weight)
    return out2d[None]
